# baseline (speedup 1.0000x reference)
import functools

import jax
import jax.numpy as jnp
import numpy as np
from jax import lax
from jax.experimental import pallas as pl
from jax.experimental.pallas import tpu as pltpu
from jax.experimental.pallas import tpu_sc as plsc

_START_NOTE = 21
_N_KEYS = 88
_NAMES = ['unison_detune_cents', 'unison_random_cents', 'B_val',
          'string_variation_std', 'decay_tau', 'decay_k', 'tau_fast_divisor',
          'hammer_xh', 'comb_mix', 'comb_base', 'tilt_base', 'tilt_slope',
          'hammer_fc_low', 'hammer_fc_high', 'hammer_fc_v_curve', 'nw_base',
          'nw_slope', 'highpass_freq', 'highpass_power', 'lowpass_freq',
          'lowpass_power', 'prompt_n_mix', 'w_min', 'w_max',
          'aftersound_fc_base', 'reverb_wet', 'reverb_decay']
_J = len(_NAMES)

_W = np.zeros((_J, 8), dtype=np.float32)
for _i, (_kind, _scale, _bias) in enumerate([
        ('sp', 1, 0), ('sp', 1, 0), ('sp', 1, 0), ('sig', 0.05, 0),
        ('sp', 1, 0), ('sp', 1, 0), ('sp', 1, 1), ('c1', 1, 0),
        ('id', 1, 0), ('id', 1, 0), ('id', 1, 0), ('id', 1, 0),
        ('sp', 1, 20), ('sp', 1, 100), ('sp', 1, 0), ('sp', 1, 0),
        ('sp', 1, 0), ('sp', 1, 20), ('sp', 1, 0), ('sp', 1, 100),
        ('sp', 1, 0), ('sp', 1, 0), ('c2', 1, 0), ('c2', 1, 0),
        ('id', 1, 0), ('sig', 1, 0), ('sp', 1, 0.05)]):
    _col = {'sp': 0, 'sig': 1, 'id': 2, 'c1': 3, 'c2': 4}[_kind]
    _W[_i, _col] = _scale
    _W[_i, 5] = _bias
_W[26, 5] = 0.05


def _table_body(x_ref, w_ref, o_ref):
    x = x_ref[...]
    w = w_ref[...]
    sp = jax.nn.softplus(x)
    sig = jax.nn.sigmoid(x)
    o_ref[...] = (w[:, 0:1] * sp + w[:, 1:2] * sig + w[:, 2:3] * x
                  + w[:, 3:4] * jnp.clip(x, 0.01, 0.5)
                  + w[:, 4:5] * jnp.clip(x, 0.0, 1.0) + w[:, 5:6])


def _build_table(stacked, w):
    return pl.pallas_call(
        _table_body,
        out_shape=jax.ShapeDtypeStruct((_J, _N_KEYS), jnp.float32),
    )(stacked, w)


_NC, _NS, _L = 2, 16, 16
_NW = _NC * _NS
_ROWS, _COLS = 16384, 200
_SLAB = _ROWS // _NW
_CC = 4
_CHUNK = _CC * _SLAB
_N_CHUNKS = _COLS // _CC
_N_SLICES = _CHUNK // _L
_JP = (_J + 1) // 2
_TAB_PAD = _JP * _N_KEYS


def _gather_body(tab_hbm, midi_hbm, out_hbm, tab_v, midi_v0, midi_v1, out_v,
                 sem_in, sem_o0, sem_o1):
    midi_bufs = (midi_v0, midi_v1)
    wid = lax.axis_index("s") * _NC + lax.axis_index("c")
    pltpu.sync_copy(tab_hbm, tab_v)
    r0 = wid * _SLAB

    def in_copy(cc, b):
        return pltpu.make_async_copy(
            midi_hbm.at[pl.ds(cc * _CC, _CC), pl.ds(r0, _SLAB)],
            midi_bufs[b], sem_in)

    def out_copies(cc, b, sem):
        return [pltpu.make_async_copy(
                    out_v.at[b * _J + j],
                    out_hbm.at[j, pl.ds(cc * _CC, _CC), pl.ds(r0, _SLAB)],
                    sem)
                for j in range(_J)]

    in_copy(0, 0).start()
    in_copy(1, 1).start()
    sems = (sem_o0, sem_o1)

    def compute_chunk(b):
        @plsc.parallel_loop(0, _N_SLICES, 1, unroll=8)
        def slice_body(i):
            c = i // (_SLAB // _L)
            base = (i % (_SLAB // _L)) * _L
            m = midi_bufs[b][c, pl.ds(base, _L)]
            idx = jnp.minimum(jnp.maximum(m - _START_NOTE, 0), _N_KEYS - 1)
            for jj in range(_JP):
                w = plsc.load_gather(tab_v, [idx + (jj * _N_KEYS)])
                lo = plsc.bitcast(lax.shift_left(w, 16), jnp.float32)
                out_v[b * _J + 2 * jj, c, pl.ds(base, _L)] = lo
                if 2 * jj + 1 < _J:
                    hi = plsc.bitcast(
                        lax.bitwise_and(w, jnp.int32(-65536)), jnp.float32)
                    out_v[b * _J + 2 * jj + 1, c, pl.ds(base, _L)] = hi

    def pair_body(ci, _):
        for b in (0, 1):
            cc = 2 * ci + b
            in_copy(cc, b).wait()

            @pl.when(ci > 0)
            def _drain():
                for c in out_copies(0, b, sems[b]):
                    c.wait()

            compute_chunk(b)
            for c in out_copies(cc, b, sems[b]):
                c.start()

            @pl.when(ci < _N_CHUNKS // 2 - 1)
            def _prefetch():
                in_copy(cc + 2, b).start()
        return 0

    lax.fori_loop(0, _N_CHUNKS // 2, pair_body, 0)
    for b in (0, 1):
        for c in out_copies(0, b, sems[b]):
            c.wait()


@functools.partial(jax.jit, static_argnames=())
def _sc_gather(tab_flat, midi_flat):
    mesh = plsc.VectorSubcoreMesh(core_axis_name="c", subcore_axis_name="s",
                                  num_cores=_NC, num_subcores=_NS)
    return pl.kernel(
        _gather_body,
        out_type=jax.ShapeDtypeStruct((_J, _COLS, _ROWS), jnp.float32),
        mesh=mesh,
        scratch_types=[
            pltpu.VMEM((_TAB_PAD,), jnp.int32),
            pltpu.VMEM((_CC, _SLAB), jnp.int32),
            pltpu.VMEM((_CC, _SLAB), jnp.int32),
            pltpu.VMEM((2 * _J, _CC, _SLAB), jnp.float32),
            pltpu.SemaphoreType.DMA,
            pltpu.SemaphoreType.DMA,
            pltpu.SemaphoreType.DMA,
        ],
        compiler_params=pltpu.CompilerParams(needs_layout_passes=False,
                                             use_tc_tiling_on_sc=True),
    )(tab_flat, midi_flat)


def kernel(midi, params):
    stacked = jnp.stack([params[n] for n in _NAMES])
    table = _build_table(stacked, jnp.asarray(_W))
    tbits = lax.bitcast_convert_type(
        table.astype(jnp.bfloat16), jnp.uint16).astype(jnp.uint32)
    tbits = jnp.concatenate(
        [tbits, jnp.zeros((2 * _JP - _J, _N_KEYS), jnp.uint32)])
    packed = (tbits[0::2] | (tbits[1::2] << 16)).astype(jnp.int32)
    midi_t = midi.astype(jnp.int32).T
    out_t = _sc_gather(packed.reshape(-1), midi_t)
    return jnp.transpose(out_t, (0, 2, 1))


# speedup vs baseline: 1.1229x; 1.1229x over previous
import functools

import jax
import jax.numpy as jnp
import numpy as np
from jax import lax
from jax.experimental import pallas as pl
from jax.experimental.pallas import tpu as pltpu
from jax.experimental.pallas import tpu_sc as plsc

_START_NOTE = 21
_N_KEYS = 88
_NAMES = ['unison_detune_cents', 'unison_random_cents', 'B_val',
          'string_variation_std', 'decay_tau', 'decay_k', 'tau_fast_divisor',
          'hammer_xh', 'comb_mix', 'comb_base', 'tilt_base', 'tilt_slope',
          'hammer_fc_low', 'hammer_fc_high', 'hammer_fc_v_curve', 'nw_base',
          'nw_slope', 'highpass_freq', 'highpass_power', 'lowpass_freq',
          'lowpass_power', 'prompt_n_mix', 'w_min', 'w_max',
          'aftersound_fc_base', 'reverb_wet', 'reverb_decay']
_J = len(_NAMES)

_W = np.zeros((_J, 8), dtype=np.float32)
for _i, (_kind, _scale, _bias) in enumerate([
        ('sp', 1, 0), ('sp', 1, 0), ('sp', 1, 0), ('sig', 0.05, 0),
        ('sp', 1, 0), ('sp', 1, 0), ('sp', 1, 1), ('c1', 1, 0),
        ('id', 1, 0), ('id', 1, 0), ('id', 1, 0), ('id', 1, 0),
        ('sp', 1, 20), ('sp', 1, 100), ('sp', 1, 0), ('sp', 1, 0),
        ('sp', 1, 0), ('sp', 1, 20), ('sp', 1, 0), ('sp', 1, 100),
        ('sp', 1, 0), ('sp', 1, 0), ('c2', 1, 0), ('c2', 1, 0),
        ('id', 1, 0), ('sig', 1, 0), ('sp', 1, 0.05)]):
    _col = {'sp': 0, 'sig': 1, 'id': 2, 'c1': 3, 'c2': 4}[_kind]
    _W[_i, _col] = _scale
    _W[_i, 5] = _bias
_W[26, 5] = 0.05


def _table_body(x_ref, w_ref, o_ref):
    x = x_ref[...]
    w = w_ref[...]
    sp = jax.nn.softplus(x)
    sig = jax.nn.sigmoid(x)
    o_ref[...] = (w[:, 0:1] * sp + w[:, 1:2] * sig + w[:, 2:3] * x
                  + w[:, 3:4] * jnp.clip(x, 0.01, 0.5)
                  + w[:, 4:5] * jnp.clip(x, 0.0, 1.0) + w[:, 5:6])


def _build_table(stacked, w):
    return pl.pallas_call(
        _table_body,
        out_shape=jax.ShapeDtypeStruct((_J, _N_KEYS), jnp.float32),
    )(stacked, w)


_NC, _NS, _L = 2, 16, 16
_NW = _NC * _NS
_ROWS, _COLS = 16384, 200
_SLAB = _ROWS // _NW
_CC = 4
_CHUNK = _CC * _SLAB
_N_CHUNKS = _COLS // _CC
_N_SLICES = _CHUNK // _L
_JP = (_J + 1) // 2
_TAB_PAD = _JP * _N_KEYS


def _gather_body(tab_hbm, midi_hbm, out_hbm, tab_v, midi_v0, midi_v1, out_v,
                 sem_in, sem_o0, sem_o1):
    midi_bufs = (midi_v0, midi_v1)
    wid = lax.axis_index("s") * _NC + lax.axis_index("c")
    pltpu.sync_copy(tab_hbm, tab_v)
    r0 = wid * _SLAB

    def in_copy(cc, b):
        return pltpu.make_async_copy(
            midi_hbm.at[pl.ds(cc * _CC, _CC), pl.ds(r0, _SLAB)],
            midi_bufs[b], sem_in)

    def out_copies(cc, b, sem):
        return [pltpu.make_async_copy(
                    out_v.at[b * _J + j],
                    out_hbm.at[j, pl.ds(cc * _CC, _CC), pl.ds(r0, _SLAB)],
                    sem)
                for j in range(_J)]

    in_copy(0, 0).start()
    in_copy(1, 1).start()
    sems = (sem_o0, sem_o1)

    def compute_chunk(b):
        @plsc.parallel_loop(0, _N_SLICES, 1, unroll=4)
        def slice_body(i):
            c = i // (_SLAB // _L)
            base = (i % (_SLAB // _L)) * _L
            m = midi_bufs[b][c, pl.ds(base, _L)]
            idx = jnp.minimum(jnp.maximum(m - _START_NOTE, 0), _N_KEYS - 1)
            for jj in range(_JP):
                w = plsc.load_gather(tab_v, [idx + (jj * _N_KEYS)])
                lo = plsc.bitcast(lax.shift_left(w, 16), jnp.float32)
                out_v[b * _J + 2 * jj, c, pl.ds(base, _L)] = lo
                if 2 * jj + 1 < _J:
                    hi = plsc.bitcast(
                        lax.bitwise_and(w, jnp.int32(-65536)), jnp.float32)
                    out_v[b * _J + 2 * jj + 1, c, pl.ds(base, _L)] = hi

    def pair_body(ci, _):
        for b in (0, 1):
            cc = 2 * ci + b
            in_copy(cc, b).wait()

            @pl.when(ci > 0)
            def _drain():
                for c in out_copies(0, b, sems[b]):
                    c.wait()

            compute_chunk(b)
            for c in out_copies(cc, b, sems[b]):
                c.start()

            @pl.when(ci < _N_CHUNKS // 2 - 1)
            def _prefetch():
                in_copy(cc + 2, b).start()
        return 0

    lax.fori_loop(0, _N_CHUNKS // 2, pair_body, 0)
    for b in (0, 1):
        for c in out_copies(0, b, sems[b]):
            c.wait()


@functools.partial(jax.jit, static_argnames=())
def _sc_gather(tab_flat, midi_flat):
    mesh = plsc.VectorSubcoreMesh(core_axis_name="c", subcore_axis_name="s",
                                  num_cores=_NC, num_subcores=_NS)
    return pl.kernel(
        _gather_body,
        out_type=jax.ShapeDtypeStruct((_J, _COLS, _ROWS), jnp.float32),
        mesh=mesh,
        scratch_types=[
            pltpu.VMEM((_TAB_PAD,), jnp.int32),
            pltpu.VMEM((_CC, _SLAB), jnp.int32),
            pltpu.VMEM((_CC, _SLAB), jnp.int32),
            pltpu.VMEM((2 * _J, _CC, _SLAB), jnp.float32),
            pltpu.SemaphoreType.DMA,
            pltpu.SemaphoreType.DMA,
            pltpu.SemaphoreType.DMA,
        ],
        compiler_params=pltpu.CompilerParams(needs_layout_passes=False,
                                             use_tc_tiling_on_sc=True),
    )(tab_flat, midi_flat)


def kernel(midi, params):
    stacked = jnp.stack([params[n] for n in _NAMES])
    table = _build_table(stacked, jnp.asarray(_W))
    tbits = lax.bitcast_convert_type(
        table.astype(jnp.bfloat16), jnp.uint16).astype(jnp.uint32)
    tbits = jnp.concatenate(
        [tbits, jnp.zeros((2 * _JP - _J, _N_KEYS), jnp.uint32)])
    packed = (tbits[0::2] | (tbits[1::2] << 16)).astype(jnp.int32)
    midi_t = midi.astype(jnp.int32).T
    out_t = _sc_gather(packed.reshape(-1), midi_t)
    return jnp.transpose(out_t, (0, 2, 1))
